# baseline (speedup 1.0000x reference)
import functools

import jax
import jax.numpy as jnp
from jax import lax
from jax.experimental import pallas as pl
from jax.experimental.pallas import tpu as pltpu
import jax.experimental.pallas.tpu_sc as plsc

N_TOK = 2048
D_IN = 2048
D_H = 4096
D_OUT = 2048
R_H = 128
E = 3

TILE = 128
NT = N_TOK // TILE + (E - 1)
NPAD = NT * TILE
HBS1 = 512
HB1 = D_H // HBS1
NOB = 256
OB = D_OUT // NOB

SC_NC = 2
SC_NS = 16
SC_NW = SC_NC * SC_NS
ROWS_PER_W = N_TOK // SC_NW
ROW_CHUNK = 16
N_CHUNK = ROWS_PER_W // ROW_CHUNK


def _plan_body(x_ref, rw1_ref, rb1_ref, rw2_ref, rb2_ref,
               pos_ref, eot_ref, stats_ref):
    xv = x_ref[...]
    h = jnp.maximum(
        jnp.dot(xv, rw1_ref[...], preferred_element_type=jnp.float32)
        + rb1_ref[...], 0.0)
    logits = (jnp.dot(h, rw2_ref[...], preferred_element_type=jnp.float32)
              + rb2_ref[...])
    l0 = logits[:, 0:1]
    l1 = logits[:, 1:2]
    l2 = logits[:, 2:3]
    sel = jnp.where(l1 > l0, 1, 0)
    sel = jnp.where(l2 > jnp.maximum(l0, l1), 2, sel)

    lane = lax.broadcasted_iota(jnp.int32, (N_TOK, 128), 1)
    onehot = (lane == sel).astype(jnp.float32)
    tri128 = (lax.broadcasted_iota(jnp.int32, (TILE, TILE), 0)
              >= lax.broadcasted_iota(jnp.int32, (TILE, TILE), 1)
              ).astype(jnp.float32)
    prefix = jnp.zeros((1, 128), jnp.float32)
    rank_rows = []
    for k in range(N_TOK // TILE):
        chunk = onehot[k * TILE:(k + 1) * TILE, :]
        within = jnp.dot(tri128, chunk, preferred_element_type=jnp.float32)
        rank_rows.append(within + prefix)
        prefix = prefix + within[TILE - 1:TILE, :]
    ranks = jnp.concatenate(rank_rows, axis=0)

    c0 = ranks[N_TOK - 1, 0].astype(jnp.int32)
    c1 = ranks[N_TOK - 1, 1].astype(jnp.int32)
    c2 = ranks[N_TOK - 1, 2].astype(jnp.int32)
    off1 = ((c0 + TILE - 1) // TILE) * TILE
    off2 = off1 + ((c1 + TILE - 1) // TILE) * TILE

    ranksel = jnp.sum(ranks * onehot, axis=1, keepdims=True)
    offsel = jnp.where(sel == 0, 0, jnp.where(sel == 1, off1, off2))
    pos = offsel + ranksel.astype(jnp.int32) - 1
    pos_ref[...] = jnp.broadcast_to(pos, (N_TOK, 128))

    lane8 = lax.broadcasted_iota(jnp.int32, (8, 128), 1)
    tstart = lane8 * TILE
    eot_ref[...] = ((tstart >= off1).astype(jnp.int32)
                    + (tstart >= off2).astype(jnp.int32))
    stats_ref[...] = jnp.where(
        lane8 == 0, c0, jnp.where(lane8 == 1, c1,
                                  jnp.where(lane8 == 2, c2, 0)))


def _run_plan(x, rw1, rb1, rw2p, rb2p):
    return pl.pallas_call(
        _plan_body,
        out_shape=(
            jax.ShapeDtypeStruct((N_TOK, 128), jnp.int32),
            jax.ShapeDtypeStruct((8, 128), jnp.int32),
            jax.ShapeDtypeStruct((8, 128), jnp.int32),
        ),
    )(x, rw1, rb1, rw2p, rb2p)


@functools.lru_cache(maxsize=None)
def _sc_kernels():
    mesh = plsc.VectorSubcoreMesh(core_axis_name="c", subcore_axis_name="s",
                                  num_cores=SC_NC, num_subcores=SC_NS)

    scatter_scratch = [
        pltpu.VMEM((N_CHUNK, ROW_CHUNK), jnp.int32),
        pltpu.VMEM((ROW_CHUNK, D_IN), jnp.float32),
        pltpu.VMEM((ROW_CHUNK, D_IN), jnp.float32),
        pltpu.SemaphoreType.DMA,
        pltpu.SemaphoreType.DMA,
        pltpu.SemaphoreType.DMA,
        pltpu.SemaphoreType.DMA,
    ]

    @functools.partial(
        pl.kernel,
        out_type=jax.ShapeDtypeStruct((NPAD, D_IN), jnp.float32),
        mesh=mesh,
        scratch_types=scatter_scratch,
    )
    def sc_scatter(x_hbm, pos3_hbm, xs_hbm, idx_m, rows_a, rows_b,
                   sem_ra, sem_rb, sem_wa, sem_wb):
        wid = lax.axis_index("s") * SC_NC + lax.axis_index("c")
        base = wid * ROWS_PER_W
        pltpu.sync_copy(pos3_hbm.at[wid], idx_m)
        bufs = [(rows_a, sem_ra, sem_wa), (rows_b, sem_rb, sem_wb)]

        def rd(c):
            buf, rs, _ = bufs[c % 2]
            return pltpu.async_copy(
                x_hbm.at[pl.ds(base + c * ROW_CHUNK, ROW_CHUNK)], buf, rs)

        def wr(c):
            buf, _, ws = bufs[c % 2]
            return pltpu.async_copy(buf, xs_hbm.at[idx_m.at[c]], ws)

        r0, r1 = rd(0), rd(1)
        r0.wait()
        w0 = wr(0)
        r1.wait()
        w1 = wr(1)
        w0.wait()
        r2 = rd(2)
        w1.wait()
        r3 = rd(3)
        r2.wait()
        w2 = wr(2)
        r3.wait()
        w3 = wr(3)
        w2.wait()
        w3.wait()

    @functools.partial(
        pl.kernel,
        out_type=jax.ShapeDtypeStruct((N_TOK, D_OUT), jnp.float32),
        mesh=mesh,
        scratch_types=scatter_scratch,
    )
    def sc_gather(ys_hbm, pos3_hbm, out_hbm, idx_m, rows_a, rows_b,
                  sem_ra, sem_rb, sem_wa, sem_wb):
        wid = lax.axis_index("s") * SC_NC + lax.axis_index("c")
        base = wid * ROWS_PER_W
        pltpu.sync_copy(pos3_hbm.at[wid], idx_m)
        bufs = [(rows_a, sem_ra, sem_wa), (rows_b, sem_rb, sem_wb)]

        def rd(c):
            buf, rs, _ = bufs[c % 2]
            return pltpu.async_copy(ys_hbm.at[idx_m.at[c]], buf, rs)

        def wr(c):
            buf, _, ws = bufs[c % 2]
            return pltpu.async_copy(
                buf, out_hbm.at[pl.ds(base + c * ROW_CHUNK, ROW_CHUNK)], ws)

        r0, r1 = rd(0), rd(1)
        r0.wait()
        w0 = wr(0)
        r1.wait()
        w1 = wr(1)
        w0.wait()
        r2 = rd(2)
        w1.wait()
        r3 = rd(3)
        r2.wait()
        w2 = wr(2)
        r3.wait()
        w3 = wr(3)
        w2.wait()
        w3.wait()

    return sc_scatter, sc_gather


def _fc1_body(eot_ref, x_hbm, w1_hbm, b1_ref, h_ref, xv_ref, wbuf_ref,
              wsem, xsem):
    hb = pl.program_id(0)

    def w1_slab_start(i, slot):
        for e in range(E):
            pltpu.make_async_copy(
                w1_hbm.at[e, :, pl.ds(i * HBS1, HBS1)],
                wbuf_ref.at[slot, e], wsem.at[slot]).start()

    def w1_slab_wait(slot):
        pltpu.make_async_copy(
            w1_hbm.at[:, :, pl.ds(0, HBS1)], wbuf_ref.at[slot],
            wsem.at[slot]).wait()

    @pl.when(hb == 0)
    def _():
        for q in range(4):
            pltpu.make_async_copy(
                x_hbm.at[pl.ds(q * (NPAD // 4), NPAD // 4)],
                xv_ref.at[pl.ds(q * (NPAD // 4), NPAD // 4)], xsem).start()
        w1_slab_start(0, 0)
        w1_slab_start(1, 1)
        pltpu.make_async_copy(x_hbm, xv_ref, xsem).wait()

    cur = lax.rem(hb, 2)

    @pl.when((hb >= 1) & (hb + 1 < HB1))
    def _():
        w1_slab_start(hb + 1, lax.rem(hb + 1, 2))

    w1_slab_wait(cur)

    for t in range(NT):
        e = eot_ref[t]
        xv = xv_ref[t * TILE:(t + 1) * TILE, :]
        hv = (jnp.dot(xv, wbuf_ref[cur, e], preferred_element_type=jnp.float32)
              + b1_ref[e])
        h_ref[t * TILE:(t + 1) * TILE, :] = (
            jnp.maximum(hv, 0.0).astype(jnp.bfloat16))


def _run_fc1(eot, xs, ew1, eb1r):
    grid_spec = pltpu.PrefetchScalarGridSpec(
        num_scalar_prefetch=1,
        grid=(HB1,),
        in_specs=[
            pl.BlockSpec(memory_space=pl.ANY),
            pl.BlockSpec(memory_space=pl.ANY),
            pl.BlockSpec((E, 1, HBS1), lambda hb, eot: (0, 0, hb)),
        ],
        out_specs=pl.BlockSpec((NPAD, HBS1), lambda hb, eot: (0, hb)),
        scratch_shapes=[pltpu.VMEM((NPAD, D_IN), jnp.float32),
                        pltpu.VMEM((2, E, D_IN, HBS1), jnp.float32),
                        pltpu.SemaphoreType.DMA((2,)),
                        pltpu.SemaphoreType.DMA],
    )
    return pl.pallas_call(
        _fc1_body,
        grid_spec=grid_spec,
        out_shape=jax.ShapeDtypeStruct((NPAD, D_H), jnp.bfloat16),
    )(eot, xs, ew1, eb1r)


def _fc2_body(eot_ref, h_hbm, w2_hbm, b2_ref, y_ref, hv_ref, wbuf_ref,
              w2b_ref, wsem, hsem):
    ob = pl.program_id(0)

    def w2_slab_start(i, slot):
        for e in range(E):
            pltpu.make_async_copy(
                w2_hbm.at[e, :, pl.ds(i * NOB, NOB)],
                wbuf_ref.at[slot, e], wsem.at[slot]).start()

    def w2_slab_wait(slot):
        pltpu.make_async_copy(
            w2_hbm.at[:, :, pl.ds(0, NOB)], wbuf_ref.at[slot],
            wsem.at[slot]).wait()

    @pl.when(ob == 0)
    def _():
        for q in range(4):
            pltpu.make_async_copy(
                h_hbm.at[pl.ds(q * (NPAD // 4), NPAD // 4)],
                hv_ref.at[pl.ds(q * (NPAD // 4), NPAD // 4)], hsem).start()
        w2_slab_start(0, 0)
        w2_slab_start(1, 1)
        pltpu.make_async_copy(h_hbm, hv_ref, hsem).wait()

    cur = lax.rem(ob, 2)

    @pl.when((ob >= 1) & (ob + 1 < OB))
    def _():
        w2_slab_start(ob + 1, lax.rem(ob + 1, 2))

    w2_slab_wait(cur)

    w2b_ref[...] = wbuf_ref[cur].astype(jnp.bfloat16)
    for t in range(NT):
        e = eot_ref[t]
        hv = hv_ref[t * TILE:(t + 1) * TILE, :]
        y_ref[t * TILE:(t + 1) * TILE, :] = (
            jnp.dot(hv, w2b_ref[e], preferred_element_type=jnp.float32)
            + b2_ref[e])


def _run_fc2(eot, h, ew2, eb2r):
    grid_spec = pltpu.PrefetchScalarGridSpec(
        num_scalar_prefetch=1,
        grid=(OB,),
        in_specs=[
            pl.BlockSpec(memory_space=pl.ANY),
            pl.BlockSpec(memory_space=pl.ANY),
            pl.BlockSpec((E, 1, NOB), lambda ob, eot: (0, 0, ob)),
        ],
        out_specs=pl.BlockSpec((NPAD, NOB), lambda ob, eot: (0, ob)),
        scratch_shapes=[pltpu.VMEM((NPAD, D_H), jnp.bfloat16),
                        pltpu.VMEM((2, E, D_H, NOB), jnp.float32),
                        pltpu.VMEM((E, D_H, NOB), jnp.bfloat16),
                        pltpu.SemaphoreType.DMA((2,)),
                        pltpu.SemaphoreType.DMA],
    )
    return pl.pallas_call(
        _fc2_body,
        grid_spec=grid_spec,
        out_shape=jax.ShapeDtypeStruct((NPAD, D_OUT), jnp.float32),
    )(eot, h, ew2, eb2r)


def kernel(x, router_W1, router_b1, router_W2, router_b2,
           expert_W1, expert_b1, expert_W2, expert_b2):
    rw2p = jnp.pad(router_W2, ((0, 0), (0, 128 - E)))
    rb2p = jnp.pad(router_b2, (0, 128 - E)).reshape(1, 128)
    rb1r = router_b1.reshape(1, R_H)

    pos2d, eot2d, stats2d = _run_plan(x, router_W1, rb1r, rw2p, rb2p)
    pos = pos2d[:, 0]
    eot = eot2d[0, :NT]
    stats = stats2d[0, :E]

    sc_scatter, sc_gather = _sc_kernels()
    pos3 = pos.reshape(SC_NW, N_CHUNK, ROW_CHUNK)
    xs = sc_scatter(x, pos3)
    eb1r = expert_b1.reshape(E, 1, D_H)
    eb2r = expert_b2.reshape(E, 1, D_OUT)
    h = _run_fc1(eot, xs, expert_W1, eb1r)
    ys = _run_fc2(eot, h, expert_W2, eb2r)
    outputs = sc_gather(ys, pos3)
    return outputs, stats
